# baseline (speedup 1.0000x reference)
import jax
import jax.numpy as jnp
import numpy as np
from jax import lax
from jax.experimental import pallas as pl
from jax.experimental.pallas import tpu as pltpu
from jax.experimental.pallas import tpu_sc as plsc

VOCAB_ = 1000000
EMBED_ = 64
BATCH_ = 4096
SEQ_ = 200

NC = 2
NS = 16
LANES = 16
NW = NC * NS

N_ROWS = BATCH_ * SEQ_
PER_W = N_ROWS // NW
CHUNK = SEQ_
N_CHUNKS = PER_W // CHUNK
SEQS_PER_CHUNK = CHUNK // SEQ_
NBUF = 8
assert N_CHUNKS % NBUF == 0


def _pos_encoding():
    position = np.arange(SEQ_, dtype=np.float32)[:, None]
    div_term = np.exp(
        np.arange(0, EMBED_, 2, dtype=np.float32) * (-np.log(10000.0) / EMBED_)
    )
    pe = np.zeros((SEQ_, EMBED_), dtype=np.float32)
    pe[:, 0::2] = np.sin(position * div_term)
    pe[:, 1::2] = np.cos(position * div_term)
    return jnp.asarray(pe)


def _sc_body(x_hbm, pe_hbm, table_hbm, out_hbm, idx_v, rows_v, pe_v,
             gsem, osem, isem):
    wid = lax.axis_index("s") * NC + lax.axis_index("c")
    base = wid * PER_W

    pltpu.sync_copy(pe_hbm, pe_v)

    for b in range(NBUF):
        off = base + b * CHUNK
        pltpu.sync_copy(x_hbm.at[pl.ds(off, CHUNK)], idx_v.at[b])
        pltpu.async_copy(table_hbm.at[idx_v.at[b]], rows_v.at[b], gsem.at[b])

    @pl.loop(0, N_CHUNKS, step=NBUF)
    def _ring(g):
        for b in range(NBUF):
            gg = g + b
            pltpu.make_async_copy(
                table_hbm.at[idx_v.at[b]], rows_v.at[b], gsem.at[b]
            ).wait()

            @pl.when(gg + NBUF < N_CHUNKS)
            def _prefetch_idx():
                pltpu.async_copy(
                    x_hbm.at[pl.ds(base + (gg + NBUF) * CHUNK, CHUNK)],
                    idx_v.at[b], isem.at[b]
                )

            @pl.loop(0, SEQ_)
            def _pos(p):
                for v in range(EMBED_ // LANES):
                    pv = pe_v[p, pl.ds(v * LANES, LANES)]
                    for k in range(SEQS_PER_CHUNK):
                        r = p + k * SEQ_
                        rows_v[b, r, pl.ds(v * LANES, LANES)] = (
                            rows_v[b, r, pl.ds(v * LANES, LANES)] + pv
                        )

            pltpu.async_copy(
                rows_v.at[b], out_hbm.at[pl.ds(base + gg * CHUNK, CHUNK)],
                osem.at[b]
            )

            pb = (b - 1) % NBUF
            pgg = gg - 1 + NBUF

            @pl.when(jnp.logical_and(pgg >= NBUF, pgg < N_CHUNKS))
            def _refill():
                pltpu.make_async_copy(
                    rows_v.at[pb],
                    out_hbm.at[pl.ds(base + (pgg - NBUF) * CHUNK, CHUNK)],
                    osem.at[pb],
                ).wait()
                pltpu.make_async_copy(
                    x_hbm.at[pl.ds(0, CHUNK)], idx_v.at[pb], isem.at[pb]
                ).wait()
                pltpu.async_copy(
                    table_hbm.at[idx_v.at[pb]], rows_v.at[pb], gsem.at[pb]
                )

    for gg in range(N_CHUNKS - NBUF, N_CHUNKS):
        b = gg % NBUF
        pltpu.make_async_copy(
            rows_v.at[b], out_hbm.at[pl.ds(base + gg * CHUNK, CHUNK)], osem.at[b]
        ).wait()


@jax.jit
def _embed(x, table, pe):
    xf = x.reshape(N_ROWS).astype(jnp.int32)
    mesh = plsc.VectorSubcoreMesh(core_axis_name="c", subcore_axis_name="s")
    out = pl.kernel(
        _sc_body,
        out_type=jax.ShapeDtypeStruct((N_ROWS, EMBED_), jnp.float32),
        mesh=mesh,
        scratch_types=[
            pltpu.VMEM((NBUF, CHUNK), jnp.int32),
            pltpu.VMEM((NBUF, CHUNK, EMBED_), jnp.float32),
            pltpu.VMEM((SEQ_, EMBED_), jnp.float32),
            pltpu.SemaphoreType.DMA((NBUF,)),
            pltpu.SemaphoreType.DMA((NBUF,)),
            pltpu.SemaphoreType.DMA((NBUF,)),
        ],
        compiler_params=pltpu.CompilerParams(
            use_tc_tiling_on_sc=False, needs_layout_passes=False
        ),
    )(xf, pe, table)
    return out.reshape(BATCH_, SEQ_, EMBED_)


def kernel(x, table):
    return _embed(x, table, _pos_encoding())


# speedup vs baseline: 1.0022x; 1.0022x over previous
import jax
import jax.numpy as jnp
import numpy as np
from jax import lax
from jax.experimental import pallas as pl
from jax.experimental.pallas import tpu as pltpu
from jax.experimental.pallas import tpu_sc as plsc

VOCAB_ = 1000000
EMBED_ = 64
BATCH_ = 4096
SEQ_ = 200

NC = 2
NS = 16
LANES = 16
NW = NC * NS

N_ROWS = BATCH_ * SEQ_
PER_W = N_ROWS // NW
CHUNK = 2 * SEQ_
N_CHUNKS = PER_W // CHUNK
SEQS_PER_CHUNK = CHUNK // SEQ_
NBUF = 4
assert N_CHUNKS % NBUF == 0


def _pos_encoding():
    position = np.arange(SEQ_, dtype=np.float32)[:, None]
    div_term = np.exp(
        np.arange(0, EMBED_, 2, dtype=np.float32) * (-np.log(10000.0) / EMBED_)
    )
    pe = np.zeros((SEQ_, EMBED_), dtype=np.float32)
    pe[:, 0::2] = np.sin(position * div_term)
    pe[:, 1::2] = np.cos(position * div_term)
    return jnp.asarray(pe)


def _sc_body(x_hbm, pe_hbm, table_hbm, out_hbm, idx_v, rows_v, pe_v,
             gsem, osem, isem):
    wid = lax.axis_index("s") * NC + lax.axis_index("c")
    base = wid * PER_W

    pltpu.sync_copy(pe_hbm, pe_v)

    for b in range(NBUF):
        off = base + b * CHUNK
        pltpu.sync_copy(x_hbm.at[pl.ds(off, CHUNK)], idx_v.at[b])
        pltpu.async_copy(table_hbm.at[idx_v.at[b]], rows_v.at[b], gsem.at[b])

    @pl.loop(0, N_CHUNKS, step=NBUF)
    def _ring(g):
        for b in range(NBUF):
            gg = g + b
            pltpu.make_async_copy(
                table_hbm.at[idx_v.at[b]], rows_v.at[b], gsem.at[b]
            ).wait()

            @pl.when(gg + NBUF < N_CHUNKS)
            def _prefetch_idx():
                pltpu.async_copy(
                    x_hbm.at[pl.ds(base + (gg + NBUF) * CHUNK, CHUNK)],
                    idx_v.at[b], isem.at[b]
                )

            @pl.loop(0, SEQ_)
            def _pos(p):
                for v in range(EMBED_ // LANES):
                    pv = pe_v[p, pl.ds(v * LANES, LANES)]
                    for k in range(SEQS_PER_CHUNK):
                        r = p + k * SEQ_
                        rows_v[b, r, pl.ds(v * LANES, LANES)] = (
                            rows_v[b, r, pl.ds(v * LANES, LANES)] + pv
                        )

            pltpu.async_copy(
                rows_v.at[b], out_hbm.at[pl.ds(base + gg * CHUNK, CHUNK)],
                osem.at[b]
            )

            pb = (b - 1) % NBUF
            pgg = gg - 1 + NBUF

            @pl.when(jnp.logical_and(pgg >= NBUF, pgg < N_CHUNKS))
            def _refill():
                pltpu.make_async_copy(
                    rows_v.at[pb],
                    out_hbm.at[pl.ds(base + (pgg - NBUF) * CHUNK, CHUNK)],
                    osem.at[pb],
                ).wait()
                pltpu.make_async_copy(
                    x_hbm.at[pl.ds(0, CHUNK)], idx_v.at[pb], isem.at[pb]
                ).wait()
                pltpu.async_copy(
                    table_hbm.at[idx_v.at[pb]], rows_v.at[pb], gsem.at[pb]
                )

    for gg in range(N_CHUNKS - NBUF, N_CHUNKS):
        b = gg % NBUF
        pltpu.make_async_copy(
            rows_v.at[b], out_hbm.at[pl.ds(base + gg * CHUNK, CHUNK)], osem.at[b]
        ).wait()


@jax.jit
def _embed(x, table, pe):
    xf = x.reshape(N_ROWS).astype(jnp.int32)
    mesh = plsc.VectorSubcoreMesh(core_axis_name="c", subcore_axis_name="s")
    out = pl.kernel(
        _sc_body,
        out_type=jax.ShapeDtypeStruct((N_ROWS, EMBED_), jnp.float32),
        mesh=mesh,
        scratch_types=[
            pltpu.VMEM((NBUF, CHUNK), jnp.int32),
            pltpu.VMEM((NBUF, CHUNK, EMBED_), jnp.float32),
            pltpu.VMEM((SEQ_, EMBED_), jnp.float32),
            pltpu.SemaphoreType.DMA((NBUF,)),
            pltpu.SemaphoreType.DMA((NBUF,)),
            pltpu.SemaphoreType.DMA((NBUF,)),
        ],
        compiler_params=pltpu.CompilerParams(
            use_tc_tiling_on_sc=False, needs_layout_passes=False
        ),
    )(xf, pe, table)
    return out.reshape(BATCH_, SEQ_, EMBED_)


def kernel(x, table):
    return _embed(x, table, _pos_encoding())
